# baseline (speedup 1.0000x reference)
import functools

import jax
import jax.numpy as jnp
from jax import lax
from jax.experimental import pallas as pl
from jax.experimental.pallas import tpu as pltpu
from jax.experimental.pallas import tpu_sc as plsc

B, V = 32, 1_000_000
D = 64
NB = V // D
CW = 8192
NCHUNK = -(-V // CW)
BM_W = NCHUNK * (CW // D)
NSEL = 80
NC = 64
_EPS = 1e-5
_IBIG = 2**30


def _bm_body(x_ref, bm_ref):
    g = pl.program_id(0)
    x = x_ref[...]
    col = g * CW + jax.lax.broadcasted_iota(jnp.int32, (B, CW), 1)
    x = jnp.where(col < V, x, -jnp.inf)
    bm_ref[...] = jnp.max(x.reshape(B, CW // D, D), axis=-1)


def _block_maxes(logits):
    return pl.pallas_call(
        _bm_body,
        grid=(NCHUNK,),
        in_specs=[pl.BlockSpec((B, CW), lambda g: (0, g))],
        out_specs=pl.BlockSpec((B, CW // D), lambda g: (0, g)),
        out_shape=jax.ShapeDtypeStruct((B, BM_W), jnp.float32),
    )(logits)


def _sel_body(bm_ref, sel_ref, x_ref):
    x_ref[...] = bm_ref[...]
    col = jax.lax.broadcasted_iota(jnp.int32, (B, BM_W), 1)
    slot = jax.lax.broadcasted_iota(jnp.int32, (B, NSEL), 1)

    def step(i, sel_acc):
        x = x_ref[...]
        m = jnp.max(x, axis=1, keepdims=True)
        cand = jnp.where(x >= m, col, _IBIG)
        gid = jnp.min(cand, axis=1, keepdims=True)
        x_ref[...] = jnp.where(cand == gid, -jnp.inf, x)
        return jnp.where(slot == i, gid, sel_acc)

    sel_ref[...] = jax.lax.fori_loop(0, NSEL, step,
                                     jnp.zeros((B, NSEL), jnp.int32))


def _select_blocks(bm):
    return pl.pallas_call(
        _sel_body,
        out_shape=jax.ShapeDtypeStruct((B, NSEL), jnp.int32),
        scratch_shapes=[pltpu.VMEM((B, BM_W), jnp.float32)],
    )(bm)


def _cand_body(g_ref, sel_ref, cv_ref, ci_ref, x_ref, gi_ref):
    x_ref[...] = g_ref[...]
    lane = jax.lax.broadcasted_iota(jnp.int32, (B, NSEL, D), 2)
    gi_ref[...] = (sel_ref[...][:, :, None] * D + lane).reshape(B, NSEL * D)

    slot = jax.lax.broadcasted_iota(jnp.int32, (B, NC), 1)

    def step(i, acc):
        cv_acc, ci_acc = acc
        x = x_ref[...]
        gidx = gi_ref[...]
        m = jnp.max(x, axis=1, keepdims=True)
        cand = jnp.where(x >= m, gidx, _IBIG)
        gi = jnp.min(cand, axis=1, keepdims=True)
        x_ref[...] = jnp.where(cand == gi, -jnp.inf, x)
        return (jnp.where(slot == i, m, cv_acc),
                jnp.where(slot == i, gi, ci_acc))

    cv, ci = jax.lax.fori_loop(0, NC, step,
                               (jnp.zeros((B, NC), jnp.float32),
                                jnp.zeros((B, NC), jnp.int32)))
    cv_ref[...] = cv
    ci_ref[...] = ci


def _extract_candidates(gathered, sel):
    return pl.pallas_call(
        _cand_body,
        out_shape=(jax.ShapeDtypeStruct((B, NC), jnp.float32),
                   jax.ShapeDtypeStruct((B, NC), jnp.int32)),
        scratch_shapes=[pltpu.VMEM((B, NSEL * D), jnp.float32),
                        pltpu.VMEM((B, NSEL * D), jnp.int32)],
    )(gathered, sel)


def _post_body(cv_ref, ci_ref, nz_ref, t_ref, tp_ref,
               samp_ref, ti_ref, tl_ref):
    scaled0 = cv_ref[...]
    ci = ci_ref[...]
    t = t_ref[...]
    temp = jnp.where(t < _EPS, 1.0, t)
    scaled = scaled0 / temp
    iota = jax.lax.broadcasted_iota(jnp.int32, (B, NC), 1)
    kth = jnp.max(jnp.where(iota == 49, scaled, -jnp.inf), axis=1, keepdims=True)
    keepk = scaled >= kth
    m = jnp.max(scaled, axis=1, keepdims=True)
    p = jnp.where(keepk, jnp.exp(scaled - m), 0.0)
    probs = p / jnp.sum(p, axis=1, keepdims=True)
    r = jax.lax.broadcasted_iota(jnp.int32, (NC, NC), 0)
    c = jax.lax.broadcasted_iota(jnp.int32, (NC, NC), 1)
    ut = (r < c).astype(jnp.float32)
    exc = jax.lax.dot(probs, ut, preferred_element_type=jnp.float32)
    surv = keepk & (exc <= tp_ref[...])
    s_cnt = jnp.sum(surv.astype(jnp.int32), axis=1, keepdims=True)
    g = -jnp.log(-jnp.log(nz_ref[...]))
    score = jnp.where(surv, scaled + g, -jnp.inf)
    ms = jnp.max(score, axis=1, keepdims=True)
    pos = jnp.min(jnp.where(score >= ms, iota, _IBIG), axis=1, keepdims=True)
    rs = jnp.sum(jnp.where(iota == pos, ci, 0), axis=1, keepdims=True)
    greedy = jnp.sum(jnp.where(iota == 0, ci, 0), axis=1, keepdims=True)
    samp_ref[...] = jnp.where(t < _EPS, greedy, rs)
    sum_surv = jnp.sum(jnp.where(surv, jnp.exp(scaled - m), 0.0), axis=1, keepdims=True)
    logz = m + jnp.log(sum_surv)
    tl_ref[...] = jnp.where(surv[:, :8], scaled[:, :8] - logz, -jnp.inf)
    used = jnp.concatenate(
        [jnp.max(jnp.where(surv & (ci == v), 1, 0), axis=1, keepdims=True)
         for v in range(16)], axis=1)
    avail = (1 - used).astype(jnp.float32)
    r16 = jax.lax.broadcasted_iota(jnp.int32, (16, 16), 0)
    c16 = jax.lax.broadcasted_iota(jnp.int32, (16, 16), 1)
    inc = (r16 <= c16).astype(jnp.float32)
    rank = jax.lax.dot(avail, inc, preferred_element_type=jnp.float32).astype(jnp.int32)
    v16 = jax.lax.broadcasted_iota(jnp.int32, (B, 16), 1)
    fills = []
    for j in range(8):
        want = j - s_cnt + 1
        hit = (avail > 0) & (rank == want)
        fills.append(jnp.sum(jnp.where(hit, v16, 0), axis=1, keepdims=True))
    fillv = jnp.concatenate(fills, axis=1)
    slots = jax.lax.broadcasted_iota(jnp.int32, (B, 8), 1)
    ti_ref[...] = jnp.where(slots < s_cnt, ci[:, :8], fillv)


def _post(cv, ci, noise_at, temperature, top_p):
    return pl.pallas_call(
        _post_body,
        out_shape=(jax.ShapeDtypeStruct((B, 1), jnp.int32),
                   jax.ShapeDtypeStruct((B, 8), jnp.int32),
                   jax.ShapeDtypeStruct((B, 8), jnp.float32)),
    )(cv, ci, noise_at, temperature.reshape(B, 1), top_p.reshape(B, 1))


_SC_CORES, _SC_SUBCORES = 2, 16
_SC_MESH = plsc.VectorSubcoreMesh(core_axis_name="c", subcore_axis_name="s")


def _sc_wid():
    return lax.axis_index("s") * _SC_CORES + lax.axis_index("c")


@functools.partial(
    pl.kernel, mesh=_SC_MESH,
    compiler_params=pltpu.CompilerParams(needs_layout_passes=False),
    out_type=jax.ShapeDtypeStruct((B, NSEL * D), jnp.float32),
    scratch_types=[pltpu.VMEM((NSEL,), jnp.int32),
                   pltpu.VMEM((NSEL,), jnp.int32),
                   pltpu.VMEM((NSEL, 256), jnp.float32),
                   pltpu.VMEM((NSEL * D,), jnp.float32),
                   pltpu.SemaphoreType.DMA],
)
def _sc_gather_blocks(grow_hbm, goff_hbm, flat_hbm, out_hbm,
                      idx_v, off_v, rows_v, out_v, sem):
    wid = _sc_wid()
    pltpu.sync_copy(grow_hbm.at[wid], idx_v)
    pltpu.sync_copy(goff_hbm.at[wid], off_v)
    pltpu.async_copy(flat_hbm.at[idx_v], rows_v, sem).wait()
    lane = lax.iota(jnp.int32, 16)

    def body(i, _):
        e = i * 16 + lane
        jv = jax.lax.shift_right_logical(e, 6)
        offv = plsc.load_gather(off_v, [jv])
        colv = offv + (e & 63)
        val = plsc.load_gather(rows_v, [jv, colv])
        out_v[pl.ds(pl.multiple_of(i * 16, 16), 16)] = val
        return 0

    lax.fori_loop(0, NSEL * D // 16, body, 0)
    pltpu.sync_copy(out_v, out_hbm.at[wid])


@functools.partial(
    pl.kernel, mesh=_SC_MESH,
    compiler_params=pltpu.CompilerParams(needs_layout_passes=False),
    out_type=jax.ShapeDtypeStruct((B, NC), jnp.float32),
    scratch_types=[pltpu.VMEM((NC,), jnp.int32),
                   pltpu.VMEM((NC,), jnp.int32),
                   pltpu.VMEM((NC, 128), jnp.float32),
                   pltpu.VMEM((NC,), jnp.float32),
                   pltpu.SemaphoreType.DMA],
)
def _sc_gather_noise(rows_hbm, rem_hbm, nf_hbm, out_hbm,
                     idx_v, rem_v, rows_v, out_v, sem):
    wid = _sc_wid()
    pltpu.sync_copy(rows_hbm.at[wid], idx_v)
    pltpu.sync_copy(rem_hbm.at[wid], rem_v)
    pltpu.async_copy(nf_hbm.at[idx_v], rows_v, sem).wait()
    for c in range(NC // 16):
        rid = lax.iota(jnp.int32, 16) + 16 * c
        rem_c = rem_v[pl.ds(16 * c, 16)]
        out_v[pl.ds(16 * c, 16)] = plsc.load_gather(rows_v, [rid, rem_c])
    pltpu.sync_copy(out_v, out_hbm.at[wid])


def kernel(logits, temperature, top_p, noise, top_k, max_num_logprobs):
    logits = logits.astype(jnp.float32)
    bm = _block_maxes(logits)
    sel = _select_blocks(bm)
    flat = logits.reshape(B * V // 256, 256)
    start = sel * D + jnp.arange(B, dtype=jnp.int32)[:, None] * V
    gathered = _sc_gather_blocks(start // 256, start % 256, flat)
    cv, ci = _extract_candidates(gathered, sel)
    nf = noise.reshape(B * V // 128, 128)
    np_ = ci + jnp.arange(B, dtype=jnp.int32)[:, None] * V
    noise_at = _sc_gather_noise(np_ // 128, np_ % 128, nf)
    samp, ti, tl = _post(cv, ci, noise_at, temperature, top_p)
    return samp.reshape(B), ti, tl


# speedup vs baseline: 2.1957x; 2.1957x over previous
import functools

import jax
import jax.numpy as jnp
from jax import lax
from jax.experimental import pallas as pl
from jax.experimental.pallas import tpu as pltpu
from jax.experimental.pallas import tpu_sc as plsc

B, V = 32, 1_000_000
D = 64
NB = V // D
CW = 8192
NCHUNK = -(-V // CW)
BM_W = NCHUNK * (CW // D)
NSEL = 80
NC = 64
_EPS = 1e-5
_IBIG = 2**30


def _bm_body(x_ref, bm_ref):
    g = pl.program_id(0)
    x = x_ref[...]
    col = g * CW + jax.lax.broadcasted_iota(jnp.int32, (B, CW), 1)
    x = jnp.where(col < V, x, -jnp.inf)
    bm_ref[...] = jnp.max(x.reshape(B, CW // D, D), axis=-1)


def _block_maxes(logits):
    return pl.pallas_call(
        _bm_body,
        grid=(NCHUNK,),
        in_specs=[pl.BlockSpec((B, CW), lambda g: (0, g))],
        out_specs=pl.BlockSpec((B, CW // D), lambda g: (0, g)),
        out_shape=jax.ShapeDtypeStruct((B, BM_W), jnp.float32),
    )(logits)


def _sel_body(bm_ref, sel_ref, x_ref):
    x_ref[...] = bm_ref[...]
    col = jax.lax.broadcasted_iota(jnp.int32, (B, BM_W), 1)
    slot = jax.lax.broadcasted_iota(jnp.int32, (B, NSEL), 1)

    def step(i, sel_acc):
        x = x_ref[...]
        m = jnp.max(x, axis=1, keepdims=True)
        cand = jnp.where(x >= m, col, _IBIG)
        gid = jnp.min(cand, axis=1, keepdims=True)
        x_ref[...] = jnp.where(cand == gid, -jnp.inf, x)
        return jnp.where(slot == i, gid, sel_acc)

    sel_ref[...] = jax.lax.fori_loop(0, NSEL, step,
                                     jnp.zeros((B, NSEL), jnp.int32))


def _select_blocks(bm):
    return pl.pallas_call(
        _sel_body,
        out_shape=jax.ShapeDtypeStruct((B, NSEL), jnp.int32),
        scratch_shapes=[pltpu.VMEM((B, BM_W), jnp.float32)],
    )(bm)


def _cand_body(g_ref, sel_ref, cv_ref, ci_ref, x_ref, gi_ref):
    x_ref[...] = g_ref[...]
    lane = jax.lax.broadcasted_iota(jnp.int32, (B, NSEL, D), 2)
    gi_ref[...] = (sel_ref[...][:, :, None] * D + lane).reshape(B, NSEL * D)

    slot = jax.lax.broadcasted_iota(jnp.int32, (B, NC), 1)

    def step(i, acc):
        cv_acc, ci_acc = acc
        x = x_ref[...]
        gidx = gi_ref[...]
        m = jnp.max(x, axis=1, keepdims=True)
        cand = jnp.where(x >= m, gidx, _IBIG)
        gi = jnp.min(cand, axis=1, keepdims=True)
        x_ref[...] = jnp.where(cand == gi, -jnp.inf, x)
        return (jnp.where(slot == i, m, cv_acc),
                jnp.where(slot == i, gi, ci_acc))

    cv, ci = jax.lax.fori_loop(0, NC, step,
                               (jnp.zeros((B, NC), jnp.float32),
                                jnp.zeros((B, NC), jnp.int32)))
    cv_ref[...] = cv
    ci_ref[...] = ci


def _extract_candidates(gathered, sel):
    return pl.pallas_call(
        _cand_body,
        out_shape=(jax.ShapeDtypeStruct((B, NC), jnp.float32),
                   jax.ShapeDtypeStruct((B, NC), jnp.int32)),
        scratch_shapes=[pltpu.VMEM((B, NSEL * D), jnp.float32),
                        pltpu.VMEM((B, NSEL * D), jnp.int32)],
    )(gathered, sel)


def _post_body(cv_ref, ci_ref, nz_ref, t_ref, tp_ref,
               samp_ref, ti_ref, tl_ref):
    scaled0 = cv_ref[...]
    ci = ci_ref[...]
    t = t_ref[...]
    temp = jnp.where(t < _EPS, 1.0, t)
    scaled = scaled0 / temp
    iota = jax.lax.broadcasted_iota(jnp.int32, (B, NC), 1)
    kth = jnp.max(jnp.where(iota == 49, scaled, -jnp.inf), axis=1, keepdims=True)
    keepk = scaled >= kth
    m = jnp.max(scaled, axis=1, keepdims=True)
    p = jnp.where(keepk, jnp.exp(scaled - m), 0.0)
    probs = p / jnp.sum(p, axis=1, keepdims=True)
    r = jax.lax.broadcasted_iota(jnp.int32, (NC, NC), 0)
    c = jax.lax.broadcasted_iota(jnp.int32, (NC, NC), 1)
    ut = (r < c).astype(jnp.float32)
    exc = jax.lax.dot(probs, ut, preferred_element_type=jnp.float32)
    surv = keepk & (exc <= tp_ref[...])
    s_cnt = jnp.sum(surv.astype(jnp.int32), axis=1, keepdims=True)
    g = -jnp.log(-jnp.log(nz_ref[...]))
    score = jnp.where(surv, scaled + g, -jnp.inf)
    ms = jnp.max(score, axis=1, keepdims=True)
    pos = jnp.min(jnp.where(score >= ms, iota, _IBIG), axis=1, keepdims=True)
    rs = jnp.sum(jnp.where(iota == pos, ci, 0), axis=1, keepdims=True)
    greedy = jnp.sum(jnp.where(iota == 0, ci, 0), axis=1, keepdims=True)
    samp_ref[...] = jnp.where(t < _EPS, greedy, rs)
    sum_surv = jnp.sum(jnp.where(surv, jnp.exp(scaled - m), 0.0), axis=1, keepdims=True)
    logz = m + jnp.log(sum_surv)
    tl_ref[...] = jnp.where(surv[:, :8], scaled[:, :8] - logz, -jnp.inf)
    used = jnp.concatenate(
        [jnp.max(jnp.where(surv & (ci == v), 1, 0), axis=1, keepdims=True)
         for v in range(16)], axis=1)
    avail = (1 - used).astype(jnp.float32)
    r16 = jax.lax.broadcasted_iota(jnp.int32, (16, 16), 0)
    c16 = jax.lax.broadcasted_iota(jnp.int32, (16, 16), 1)
    inc = (r16 <= c16).astype(jnp.float32)
    rank = jax.lax.dot(avail, inc, preferred_element_type=jnp.float32).astype(jnp.int32)
    v16 = jax.lax.broadcasted_iota(jnp.int32, (B, 16), 1)
    fills = []
    for j in range(8):
        want = j - s_cnt + 1
        hit = (avail > 0) & (rank == want)
        fills.append(jnp.sum(jnp.where(hit, v16, 0), axis=1, keepdims=True))
    fillv = jnp.concatenate(fills, axis=1)
    slots = jax.lax.broadcasted_iota(jnp.int32, (B, 8), 1)
    ti_ref[...] = jnp.where(slots < s_cnt, ci[:, :8], fillv)


def _post(cv, ci, noise_at, temperature, top_p):
    return pl.pallas_call(
        _post_body,
        out_shape=(jax.ShapeDtypeStruct((B, 1), jnp.int32),
                   jax.ShapeDtypeStruct((B, 8), jnp.int32),
                   jax.ShapeDtypeStruct((B, 8), jnp.float32)),
    )(cv, ci, noise_at, temperature.reshape(B, 1), top_p.reshape(B, 1))


_SC_CORES, _SC_SUBCORES = 2, 16
_SC_MESH = plsc.VectorSubcoreMesh(core_axis_name="c", subcore_axis_name="s")


def _sc_wid():
    return lax.axis_index("s") * _SC_CORES + lax.axis_index("c")


@functools.partial(
    pl.kernel, mesh=_SC_MESH,
    compiler_params=pltpu.CompilerParams(needs_layout_passes=False),
    out_type=jax.ShapeDtypeStruct((B, NSEL * D), jnp.float32),
    scratch_types=[pltpu.VMEM((NSEL,), jnp.int32),
                   pltpu.VMEM((NSEL,), jnp.int32),
                   pltpu.VMEM((NSEL, 256), jnp.float32),
                   pltpu.VMEM((NSEL * D,), jnp.float32),
                   pltpu.SemaphoreType.DMA],
)
def _sc_gather_blocks(grow_hbm, goff_hbm, flat_hbm, out_hbm,
                      idx_v, off_v, rows_v, out_v, sem):
    wid = _sc_wid()
    pltpu.sync_copy(grow_hbm.at[wid], idx_v)
    pltpu.sync_copy(goff_hbm.at[wid], off_v)
    pltpu.async_copy(flat_hbm.at[idx_v], rows_v, sem).wait()
    lane = lax.iota(jnp.int32, 16)

    def body(i, _):
        e = i * 16 + lane
        jv = jax.lax.shift_right_logical(e, 6)
        offv = plsc.load_gather(off_v, [jv])
        colv = offv + (e & 63)
        val = plsc.load_gather(rows_v, [jv, colv])
        out_v[pl.ds(pl.multiple_of(i * 16, 16), 16)] = val
        return 0

    lax.fori_loop(0, NSEL * D // 16, body, 0)
    pltpu.sync_copy(out_v, out_hbm.at[wid])


@functools.partial(
    pl.kernel, mesh=_SC_MESH,
    compiler_params=pltpu.CompilerParams(needs_layout_passes=False),
    out_type=jax.ShapeDtypeStruct((B, NC), jnp.float32),
    scratch_types=[pltpu.VMEM((NC,), jnp.int32),
                   pltpu.VMEM((NC,), jnp.int32),
                   pltpu.VMEM((NC, 128), jnp.float32),
                   pltpu.VMEM((NC,), jnp.float32),
                   pltpu.SemaphoreType.DMA],
)
def _sc_gather_noise(rows_hbm, rem_hbm, nf_hbm, out_hbm,
                     idx_v, rem_v, rows_v, out_v, sem):
    wid = _sc_wid()
    pltpu.sync_copy(rows_hbm.at[wid], idx_v)
    pltpu.sync_copy(rem_hbm.at[wid], rem_v)
    pltpu.async_copy(nf_hbm.at[idx_v], rows_v, sem).wait()
    for c in range(NC // 16):
        rid = lax.iota(jnp.int32, 16) + 16 * c
        rem_c = rem_v[pl.ds(16 * c, 16)]
        out_v[pl.ds(16 * c, 16)] = plsc.load_gather(rows_v, [rid, rem_c])
    pltpu.sync_copy(out_v, out_hbm.at[wid])


def kernel(logits, temperature, top_p, noise, top_k, max_num_logprobs):
    logits = logits.astype(jnp.float32)
    bm = _block_maxes(logits)
    sel = _select_blocks(bm)
    flat = logits.reshape(B * V // 256, 256)
    start = sel * D + jnp.arange(B, dtype=jnp.int32)[:, None] * V
    gathered = _sc_gather_blocks(start // 256, start % 256, flat)
    cv, ci = _extract_candidates(gathered, sel)
    noise_at = jnp.take_along_axis(noise, ci, axis=1)
    samp, ti, tl = _post(cv, ci, noise_at, temperature, top_p)
    return samp.reshape(B), ti, tl


# speedup vs baseline: 9.9024x; 4.5098x over previous
import jax
import jax.numpy as jnp
from jax.experimental import pallas as pl
from jax.experimental.pallas import tpu as pltpu

B, V = 32, 1_000_000
D = 64
NB = V // D
CW = 8192
NCHUNK = -(-V // CW)
BM_W = NCHUNK * (CW // D)
NSEL = 80
NC = 64
_EPS = 1e-5
_IBIG = 2**30


def _bm_body(x_ref, bm_ref):
    g = pl.program_id(0)
    x = x_ref[...]
    col = g * CW + jax.lax.broadcasted_iota(jnp.int32, (B, CW), 1)
    x = jnp.where(col < V, x, -jnp.inf)
    bm_ref[...] = jnp.max(x.reshape(B, CW // D, D), axis=-1)


def _block_maxes(logits):
    return pl.pallas_call(
        _bm_body,
        grid=(NCHUNK,),
        in_specs=[pl.BlockSpec((B, CW), lambda g: (0, g))],
        out_specs=pl.BlockSpec((B, CW // D), lambda g: (0, g)),
        out_shape=jax.ShapeDtypeStruct((B, BM_W), jnp.float32),
    )(logits)


def _sel_body(bm_ref, sel_ref, x_ref):
    x_ref[...] = bm_ref[...]
    col = jax.lax.broadcasted_iota(jnp.int32, (B, BM_W), 1)
    slot = jax.lax.broadcasted_iota(jnp.int32, (B, NSEL), 1)

    def step(i, sel_acc):
        x = x_ref[...]
        m = jnp.max(x, axis=1, keepdims=True)
        cand = jnp.where(x >= m, col, _IBIG)
        gid = jnp.min(cand, axis=1, keepdims=True)
        x_ref[...] = jnp.where(cand == gid, -jnp.inf, x)
        return jnp.where(slot == i, gid, sel_acc)

    sel_ref[...] = jax.lax.fori_loop(0, NSEL, step,
                                     jnp.zeros((B, NSEL), jnp.int32))


def _select_blocks(bm):
    return pl.pallas_call(
        _sel_body,
        out_shape=jax.ShapeDtypeStruct((B, NSEL), jnp.int32),
        scratch_shapes=[pltpu.VMEM((B, BM_W), jnp.float32)],
    )(bm)


def _cand_body(g_ref, sel_ref, cv_ref, ci_ref, x_ref, gi_ref):
    x_ref[...] = g_ref[...]
    lane = jax.lax.broadcasted_iota(jnp.int32, (B, NSEL, D), 2)
    gi_ref[...] = (sel_ref[...][:, :, None] * D + lane).reshape(B, NSEL * D)

    slot = jax.lax.broadcasted_iota(jnp.int32, (B, NC), 1)

    def step(i, acc):
        cv_acc, ci_acc = acc
        x = x_ref[...]
        gidx = gi_ref[...]
        m = jnp.max(x, axis=1, keepdims=True)
        cand = jnp.where(x >= m, gidx, _IBIG)
        gi = jnp.min(cand, axis=1, keepdims=True)
        x_ref[...] = jnp.where(cand == gi, -jnp.inf, x)
        return (jnp.where(slot == i, m, cv_acc),
                jnp.where(slot == i, gi, ci_acc))

    cv, ci = jax.lax.fori_loop(0, NC, step,
                               (jnp.zeros((B, NC), jnp.float32),
                                jnp.zeros((B, NC), jnp.int32)))
    cv_ref[...] = cv
    ci_ref[...] = ci


def _extract_candidates(gathered, sel):
    return pl.pallas_call(
        _cand_body,
        out_shape=(jax.ShapeDtypeStruct((B, NC), jnp.float32),
                   jax.ShapeDtypeStruct((B, NC), jnp.int32)),
        scratch_shapes=[pltpu.VMEM((B, NSEL * D), jnp.float32),
                        pltpu.VMEM((B, NSEL * D), jnp.int32)],
    )(gathered, sel)


def _post_body(cv_ref, ci_ref, nz_ref, t_ref, tp_ref,
               samp_ref, ti_ref, tl_ref):
    scaled0 = cv_ref[...]
    ci = ci_ref[...]
    t = t_ref[...]
    temp = jnp.where(t < _EPS, 1.0, t)
    scaled = scaled0 / temp
    iota = jax.lax.broadcasted_iota(jnp.int32, (B, NC), 1)
    kth = jnp.max(jnp.where(iota == 49, scaled, -jnp.inf), axis=1, keepdims=True)
    keepk = scaled >= kth
    m = jnp.max(scaled, axis=1, keepdims=True)
    p = jnp.where(keepk, jnp.exp(scaled - m), 0.0)
    probs = p / jnp.sum(p, axis=1, keepdims=True)
    r = jax.lax.broadcasted_iota(jnp.int32, (NC, NC), 0)
    c = jax.lax.broadcasted_iota(jnp.int32, (NC, NC), 1)
    ut = (r < c).astype(jnp.float32)
    exc = jax.lax.dot(probs, ut, preferred_element_type=jnp.float32)
    surv = keepk & (exc <= tp_ref[...])
    s_cnt = jnp.sum(surv.astype(jnp.int32), axis=1, keepdims=True)
    g = -jnp.log(-jnp.log(nz_ref[...]))
    score = jnp.where(surv, scaled + g, -jnp.inf)
    ms = jnp.max(score, axis=1, keepdims=True)
    pos = jnp.min(jnp.where(score >= ms, iota, _IBIG), axis=1, keepdims=True)
    rs = jnp.sum(jnp.where(iota == pos, ci, 0), axis=1, keepdims=True)
    greedy = jnp.sum(jnp.where(iota == 0, ci, 0), axis=1, keepdims=True)
    samp_ref[...] = jnp.where(t < _EPS, greedy, rs)
    sum_surv = jnp.sum(jnp.where(surv, jnp.exp(scaled - m), 0.0), axis=1, keepdims=True)
    logz = m + jnp.log(sum_surv)
    tl_ref[...] = jnp.where(surv[:, :8], scaled[:, :8] - logz, -jnp.inf)
    used = jnp.concatenate(
        [jnp.max(jnp.where(surv & (ci == v), 1, 0), axis=1, keepdims=True)
         for v in range(16)], axis=1)
    avail = (1 - used).astype(jnp.float32)
    r16 = jax.lax.broadcasted_iota(jnp.int32, (16, 16), 0)
    c16 = jax.lax.broadcasted_iota(jnp.int32, (16, 16), 1)
    inc = (r16 <= c16).astype(jnp.float32)
    rank = jax.lax.dot(avail, inc, preferred_element_type=jnp.float32).astype(jnp.int32)
    v16 = jax.lax.broadcasted_iota(jnp.int32, (B, 16), 1)
    fills = []
    for j in range(8):
        want = j - s_cnt + 1
        hit = (avail > 0) & (rank == want)
        fills.append(jnp.sum(jnp.where(hit, v16, 0), axis=1, keepdims=True))
    fillv = jnp.concatenate(fills, axis=1)
    slots = jax.lax.broadcasted_iota(jnp.int32, (B, 8), 1)
    ti_ref[...] = jnp.where(slots < s_cnt, ci[:, :8], fillv)


def _post(cv, ci, noise_at, temperature, top_p):
    return pl.pallas_call(
        _post_body,
        out_shape=(jax.ShapeDtypeStruct((B, 1), jnp.int32),
                   jax.ShapeDtypeStruct((B, 8), jnp.int32),
                   jax.ShapeDtypeStruct((B, 8), jnp.float32)),
    )(cv, ci, noise_at, temperature.reshape(B, 1), top_p.reshape(B, 1))


def kernel(logits, temperature, top_p, noise, top_k, max_num_logprobs):
    logits = logits.astype(jnp.float32)
    bm = _block_maxes(logits)
    sel = _select_blocks(bm)
    flat = logits.reshape(B * NB, D)
    gsel = sel + jnp.arange(B, dtype=jnp.int32)[:, None] * NB
    gathered = jnp.take(flat, gsel.reshape(-1), axis=0).reshape(B, NSEL * D)
    cv, ci = _extract_candidates(gathered, sel)
    noise_at = jnp.take_along_axis(noise, ci, axis=1)
    samp, ti, tl = _post(cv, ci, noise_at, temperature, top_p)
    return samp.reshape(B), ti, tl
